# initial kernel scaffold (unmeasured)
import jax
import jax.numpy as jnp
from jax import lax
from jax.experimental import pallas as pl
from jax.experimental.pallas import tpu as pltpu


def kernel(
    x,
):
    def body(*refs):
        pass

    out_shape = jax.ShapeDtypeStruct(..., jnp.float32)
    return pl.pallas_call(body, out_shape=out_shape)(...)


# baseline (device time: 45440 ns/iter reference)
import jax
import jax.numpy as jnp
from jax import lax
from jax.experimental import pallas as pl
from jax.experimental.pallas import tpu as pltpu

N_DEV = 4


def kernel(x):
    x = x.reshape(x.shape[-2], x.shape[-1])
    m, n_total = x.shape
    n_per = n_total // N_DEV

    def body(x_ref, out_ref, acc_ref, recv_ref, send_sems, recv_sems):
        my = lax.axis_index("i")
        left = lax.rem(my + N_DEV - 1, N_DEV)
        right = lax.rem(my + 1, N_DEV)

        barrier_sem = pltpu.get_barrier_semaphore()
        for nbr in (left, right):
            pl.semaphore_signal(
                barrier_sem, inc=1,
                device_id=(nbr,), device_id_type=pl.DeviceIdType.MESH,
            )
        pl.semaphore_wait(barrier_sem, 2)

        first = lax.rem(my + N_DEV - 1, N_DEV)
        acc_ref[...] = x_ref[:, pl.ds(first * n_per, n_per)]

        for h in range(N_DEV - 1):
            rdma = pltpu.make_async_remote_copy(
                src_ref=acc_ref,
                dst_ref=recv_ref.at[h],
                send_sem=send_sems.at[h],
                recv_sem=recv_sems.at[h],
                device_id=(right,),
                device_id_type=pl.DeviceIdType.MESH,
            )
            rdma.start()
            rdma.wait()
            idx = lax.rem(my + 2 * N_DEV - h - 2, N_DEV)
            if h < N_DEV - 2:
                acc_ref[...] = recv_ref[h] + x_ref[:, pl.ds(idx * n_per, n_per)]
            else:
                out_ref[...] = recv_ref[h] + x_ref[:, pl.ds(idx * n_per, n_per)]

    return pl.pallas_call(
        body,
        out_shape=jax.ShapeDtypeStruct((m, n_per), jnp.float32),
        in_specs=[pl.BlockSpec(memory_space=pltpu.VMEM)],
        out_specs=pl.BlockSpec(memory_space=pltpu.VMEM),
        scratch_shapes=[
            pltpu.VMEM((m, n_per), jnp.float32),
            pltpu.VMEM((N_DEV - 1, m, n_per), jnp.float32),
            pltpu.SemaphoreType.DMA((N_DEV - 1,)),
            pltpu.SemaphoreType.DMA((N_DEV - 1,)),
        ],
        compiler_params=pltpu.CompilerParams(collective_id=0),
    )(x)


# device time: 24062 ns/iter; 1.8885x vs baseline; 1.8885x over previous
import jax
import jax.numpy as jnp
from jax import lax
from jax.experimental import pallas as pl
from jax.experimental.pallas import tpu as pltpu

N_DEV = 4
N_HOP = N_DEV - 1
SUBS = 2


def kernel(x):
    x = x.reshape(x.shape[-2], x.shape[-1])
    m, n_total = x.shape
    n_per = n_total // N_DEV
    m_half = m // 2
    rps = m_half // SUBS

    def body(x_ref, out_ref, acc0_ref, recv_ref, send_sems, recv_sems):
        my = lax.axis_index("i")
        left = lax.rem(my + N_DEV - 1, N_DEV)
        right = lax.rem(my + 1, N_DEV)

        barrier_sem = pltpu.get_barrier_semaphore()
        for nbr in (left, right):
            pl.semaphore_signal(
                barrier_sem, inc=1,
                device_id=(nbr,), device_id_type=pl.DeviceIdType.MESH,
            )
        pl.semaphore_wait(barrier_sem, 2)

        targets = (right, left)

        def send_chunk_idx(d, h):
            off = (N_DEV - h - 1) if d == 0 else (h + 1)
            return lax.rem(my + off, N_DEV)

        def recv_chunk_idx(d, h):
            off = (2 * N_DEV - h - 2) if d == 0 else (h + 2)
            return lax.rem(my + off, N_DEV)

        for d in range(2):
            c = send_chunk_idx(d, 0)
            acc0_ref[d] = x_ref[
                pl.ds(d * m_half, m_half), pl.ds(c * n_per, n_per)
            ]

        def make_rdma(d, h, s):
            src = (
                acc0_ref.at[d, pl.ds(s * rps, rps)]
                if h == 0
                else recv_ref.at[d, h - 1, pl.ds(s * rps, rps)]
            )
            return pltpu.make_async_remote_copy(
                src_ref=src,
                dst_ref=recv_ref.at[d, h, pl.ds(s * rps, rps)],
                send_sem=send_sems.at[d, h, s],
                recv_sem=recv_sems.at[d, h, s],
                device_id=(targets[d],),
                device_id_type=pl.DeviceIdType.MESH,
            )

        rdmas = {}
        for s in range(SUBS):
            for d in range(2):
                rdmas[(d, 0, s)] = make_rdma(d, 0, s)
                rdmas[(d, 0, s)].start()

        for h in range(N_HOP):
            for s in range(SUBS):
                for d in range(2):
                    rdmas[(d, h, s)].wait_recv()
                    rows = pl.ds(d * m_half + s * rps, rps)
                    c = recv_chunk_idx(d, h)
                    xs = x_ref[rows, pl.ds(c * n_per, n_per)]
                    if h < N_HOP - 1:
                        recv_ref[d, h, pl.ds(s * rps, rps)] = (
                            recv_ref[d, h, pl.ds(s * rps, rps)] + xs
                        )
                        rdmas[(d, h + 1, s)] = make_rdma(d, h + 1, s)
                        rdmas[(d, h + 1, s)].start()
                    else:
                        out_ref[rows, :] = (
                            recv_ref[d, h, pl.ds(s * rps, rps)] + xs
                        )

        for rdma in rdmas.values():
            rdma.wait_send()

    return pl.pallas_call(
        body,
        out_shape=jax.ShapeDtypeStruct((m, n_per), jnp.float32),
        in_specs=[pl.BlockSpec(memory_space=pltpu.VMEM)],
        out_specs=pl.BlockSpec(memory_space=pltpu.VMEM),
        scratch_shapes=[
            pltpu.VMEM((2, m_half, n_per), jnp.float32),
            pltpu.VMEM((2, N_HOP, m_half, n_per), jnp.float32),
            pltpu.SemaphoreType.DMA((2, N_HOP, SUBS)),
            pltpu.SemaphoreType.DMA((2, N_HOP, SUBS)),
        ],
        compiler_params=pltpu.CompilerParams(collective_id=0),
    )(x)
